# Initial kernel scaffold; baseline (speedup 1.0000x reference)
import jax
import jax.numpy as jnp
from jax.experimental import pallas as pl


def kernel(kv, score, ape, state_cache, positions, slot_mapping):
    raise NotImplementedError("write your pallas kernel here")


# speedup vs baseline: 139.5841x; 139.5841x over previous
import functools

import jax
import jax.numpy as jnp
from jax import lax
from jax.experimental import pallas as pl
from jax.experimental.pallas import tpu as pltpu
from jax.experimental.pallas import tpu_sc as plsc

D = 512
T_TOK = 32768
N_SLOTS = 65536
N_ROWS = 2 * N_SLOTS
APE_MASK = 127
NW = 32
SPW = N_SLOTS // NW
SEG = 2048
NSEG = T_TOK // SEG
C = 64
NCH = SPW // C


def _sc_scatter(kv, score, ape, positions, slot_mapping, out_ref):
    mesh = plsc.VectorSubcoreMesh(core_axis_name="c", subcore_axis_name="s")

    @functools.partial(
        pl.kernel,
        mesh=mesh,
        compiler_params=pltpu.CompilerParams(needs_layout_passes=False),
        scratch_types=[
            pltpu.VMEM((SEG,), jnp.int32),
            pltpu.VMEM((SEG,), jnp.int32),
            pltpu.VMEM((SEG + 16,), jnp.int32),
            pltpu.VMEM((SEG + 16,), jnp.int32),
            pltpu.VMEM((SEG + 16,), jnp.int32),
            pltpu.VMEM((SPW + 16,), jnp.int32),
            pltpu.VMEM((SPW + 16,), jnp.int32),
            pltpu.VMEM((SPW + 80,), jnp.int32),
            pltpu.VMEM((SPW + 80,), jnp.int32),
            pltpu.VMEM((SPW + 80,), jnp.int32),
            pltpu.VMEM((NCH, C), jnp.int32),
            pltpu.VMEM((NCH, C), jnp.int32),
            pltpu.VMEM((C, D), jnp.float32),
            pltpu.VMEM((C, D), jnp.float32),
            pltpu.VMEM((C, D), jnp.float32),
            pltpu.SemaphoreType.DMA,
            pltpu.SemaphoreType.DMA,
        ],
    )
    def body(kv_hbm, score_hbm, ape_hbm, pos_hbm, slot_hbm, out_hbm,
             slots_v, pos_v, hit_t, hit_sl, hit_r, win_t, win_r,
             cmp_t, cmp_r, cmp_s, idx_kv, idx_sec, kv_buf, sc_buf, ape_buf,
             gsem, ssem):
        wid = lax.axis_index("s") * 2 + lax.axis_index("c")
        base = wid * SPW
        iota = lax.iota(jnp.int32, 16)
        neg1 = jnp.full((16,), -1, jnp.int32)

        @pl.loop(0, SPW // 16)
        def _(j):
            win_t[pl.ds(j * 16, 16)] = neg1

        @pl.loop(0, NSEG)
        def _(s):
            off = s * SEG
            pltpu.sync_copy(slot_hbm.at[pl.ds(off, SEG)], slots_v)
            pltpu.sync_copy(pos_hbm.at[pl.ds(off, SEG)], pos_v)

            def scan_body(j, cur):
                sl = slots_v[pl.ds(j * 16, 16)]
                in_rng = (sl >= base) & (sl < base + SPW)
                tvec = iota + (off + j * 16)
                rvec = pos_v[pl.ds(j * 16, 16)] & APE_MASK
                pos = plsc.cumsum(in_rng.astype(jnp.int32))
                widx = jnp.where(in_rng, cur + pos - 1, SEG)
                plsc.store_scatter(hit_t, [widx], tvec)
                plsc.store_scatter(hit_sl, [widx], sl - base)
                plsc.store_scatter(hit_r, [widx], rvec)
                return cur + plsc.all_reduce_population_count(in_rng)[0]

            nhits = lax.fori_loop(0, SEG // 16, scan_body, jnp.int32(0))

            def dedup_body(jv, _):
                sv = hit_sl[pl.ds(jv * 16, 16)]
                tv = hit_t[pl.ds(jv * 16, 16)]
                rv = hit_r[pl.ds(jv * 16, 16)]
                nvalid = jnp.minimum(jnp.int32(16), nhits - jv * 16)
                keep = iota < nvalid
                for k in range(1, 16):
                    rot = sv.at[(iota + k) & 15].get(
                        mode="promise_in_bounds")
                    later = (iota + k) < nvalid
                    keep = keep & ~((rot == sv) & later)
                sidx = jnp.where(keep, sv, SPW)
                plsc.store_scatter(win_t, [sidx], tv)
                plsc.store_scatter(win_r, [sidx], rv)
                return 0

            lax.fori_loop(0, (nhits + 15) // 16, dedup_body, 0)

        DUMP = SPW + 64

        def cmp_body(j, cur):
            wt = win_t[pl.ds(j * 16, 16)]
            m = wt >= 0
            svec = iota + (base + j * 16)
            pos = plsc.cumsum(m.astype(jnp.int32))
            widx = jnp.where(m, cur + pos - 1, DUMP)
            plsc.store_scatter(cmp_t, [widx], wt)
            plsc.store_scatter(cmp_r, [widx], win_r[pl.ds(j * 16, 16)])
            plsc.store_scatter(cmp_s, [widx], svec)
            return cur + plsc.all_reduce_population_count(m)[0]

        n_win = lax.fori_loop(0, SPW // 16, cmp_body, jnp.int32(0))
        nch = (n_win + C - 1) // C
        npad = nch * C - n_win

        last_off = jnp.maximum(n_win - 1, 0)
        last_t = jnp.full((16,), cmp_t[pl.ds(last_off, 16)][0], jnp.int32)
        last_r = jnp.full((16,), cmp_r[pl.ds(last_off, 16)][0], jnp.int32)
        last_s = jnp.full((16,), cmp_s[pl.ds(last_off, 16)][0], jnp.int32)
        for jk in range(C // 16):
            m = (iota + jk * 16) < npad
            pidx = jnp.where(m, n_win + jk * 16 + iota, DUMP)
            plsc.store_scatter(cmp_t, [pidx], last_t)
            plsc.store_scatter(cmp_r, [pidx], last_r)
            plsc.store_scatter(cmp_s, [pidx], last_s)

        for j in range(SPW // 16):
            svec = cmp_s[pl.ds(j * 16, 16)]
            row, col = j // (C // 16), (j % (C // 16)) * 16
            idx_kv[row, pl.ds(col, 16)] = svec * 2
            idx_sec[row, pl.ds(col, 16)] = svec * 2 + 1

        def data_body(c, _):
            co = c * C
            g1 = pltpu.async_copy(kv_hbm.at[cmp_t.at[pl.ds(co, C)]],
                                  kv_buf, gsem)
            g2 = pltpu.async_copy(score_hbm.at[cmp_t.at[pl.ds(co, C)]],
                                  sc_buf, gsem)
            g3 = pltpu.async_copy(ape_hbm.at[cmp_r.at[pl.ds(co, C)]],
                                  ape_buf, gsem)
            g1.wait()
            s1 = pltpu.async_copy(kv_buf, out_hbm.at[idx_kv.at[c]], ssem)
            g2.wait()
            g3.wait()

            def add_body(i, _):
                for v in range(D // 16):
                    sc_buf[i, pl.ds(v * 16, 16)] = (
                        sc_buf[i, pl.ds(v * 16, 16)]
                        + ape_buf[i, pl.ds(v * 16, 16)])
                return 0

            lax.fori_loop(0, C, add_body, 0)

            s2 = pltpu.async_copy(sc_buf, out_hbm.at[idx_sec.at[c]], ssem)
            s1.wait()
            s2.wait()
            return 0

        lax.fori_loop(0, nch, data_body, 0)

    body(kv, score, ape, positions, slot_mapping, out_ref)


def kernel(kv, score, ape, state_cache, positions, slot_mapping):
    flat = state_cache.reshape(N_ROWS, D)
    ref = jax.new_ref(flat)
    _sc_scatter(kv, score, ape, positions.astype(jnp.int32),
                slot_mapping.astype(jnp.int32), ref)
    return ref[...].reshape(state_cache.shape)


# speedup vs baseline: 222.5809x; 1.5946x over previous
import functools

import jax
import jax.numpy as jnp
from jax import lax
from jax.experimental import pallas as pl
from jax.experimental.pallas import tpu as pltpu
from jax.experimental.pallas import tpu_sc as plsc

D = 512
T_TOK = 32768
N_SLOTS = 65536
APE_MASK = 127
NW = 32
SPW = N_SLOTS // NW
SEG = 2048
NSEG = T_TOK // SEG
C = 32
NCH = SPW // C


def _sc_scatter(kv, score, ape, positions, slot_mapping, out_ref):
    mesh = plsc.VectorSubcoreMesh(core_axis_name="c", subcore_axis_name="s")

    @functools.partial(
        pl.kernel,
        mesh=mesh,
        compiler_params=pltpu.CompilerParams(needs_layout_passes=False),
        scratch_types=[
            pltpu.VMEM((SEG,), jnp.int32),
            pltpu.VMEM((SEG,), jnp.int32),
            pltpu.VMEM((SEG + 16,), jnp.int32),
            pltpu.VMEM((SEG + 16,), jnp.int32),
            pltpu.VMEM((SEG + 16,), jnp.int32),
            pltpu.VMEM((SPW + 16,), jnp.int32),
            pltpu.VMEM((SPW + 16,), jnp.int32),
            pltpu.VMEM((SPW + 80,), jnp.int32),
            pltpu.VMEM((SPW + 80,), jnp.int32),
            pltpu.VMEM((SPW + 80,), jnp.int32),
            pltpu.VMEM((NCH, C), jnp.int32),
            pltpu.VMEM((C, 2 * D), jnp.float32),
            pltpu.VMEM((C, D), jnp.float32),
            pltpu.SemaphoreType.DMA,
            pltpu.SemaphoreType.DMA,
        ],
    )
    def body(kv_hbm, score_hbm, ape_hbm, pos_hbm, slot_hbm, out_hbm,
             slots_v, pos_v, hit_t, hit_sl, hit_r, win_t, win_r,
             cmp_t, cmp_r, cmp_s, idx_s, row_buf, ape_buf, gsem, ssem):
        wid = lax.axis_index("s") * 2 + lax.axis_index("c")
        base = wid * SPW
        iota = lax.iota(jnp.int32, 16)
        neg1 = jnp.full((16,), -1, jnp.int32)

        @pl.loop(0, SPW // 16)
        def _(j):
            win_t[pl.ds(j * 16, 16)] = neg1

        @pl.loop(0, NSEG)
        def _(s):
            off = s * SEG
            pltpu.sync_copy(slot_hbm.at[pl.ds(off, SEG)], slots_v)
            pltpu.sync_copy(pos_hbm.at[pl.ds(off, SEG)], pos_v)

            def scan_body(j, cur):
                sl = slots_v[pl.ds(j * 16, 16)]
                in_rng = (sl >= base) & (sl < base + SPW)
                tvec = iota + (off + j * 16)
                rvec = pos_v[pl.ds(j * 16, 16)] & APE_MASK
                pos = plsc.cumsum(in_rng.astype(jnp.int32))
                widx = jnp.where(in_rng, cur + pos - 1, SEG)
                plsc.store_scatter(hit_t, [widx], tvec)
                plsc.store_scatter(hit_sl, [widx], sl - base)
                plsc.store_scatter(hit_r, [widx], rvec)
                return cur + plsc.all_reduce_population_count(in_rng)[0]

            nhits = lax.fori_loop(0, SEG // 16, scan_body, jnp.int32(0))

            def dedup_body(jv, _):
                sv = hit_sl[pl.ds(jv * 16, 16)]
                tv = hit_t[pl.ds(jv * 16, 16)]
                rv = hit_r[pl.ds(jv * 16, 16)]
                nvalid = jnp.minimum(jnp.int32(16), nhits - jv * 16)
                keep = iota < nvalid
                for k in range(1, 16):
                    rot = sv.at[(iota + k) & 15].get(
                        mode="promise_in_bounds")
                    later = (iota + k) < nvalid
                    keep = keep & ~((rot == sv) & later)
                sidx = jnp.where(keep, sv, SPW)
                plsc.store_scatter(win_t, [sidx], tv)
                plsc.store_scatter(win_r, [sidx], rv)
                return 0

            lax.fori_loop(0, (nhits + 15) // 16, dedup_body, 0)

        DUMP = SPW + 64

        def cmp_body(j, cur):
            wt = win_t[pl.ds(j * 16, 16)]
            m = wt >= 0
            svec = iota + (base + j * 16)
            pos = plsc.cumsum(m.astype(jnp.int32))
            widx = jnp.where(m, cur + pos - 1, DUMP)
            plsc.store_scatter(cmp_t, [widx], wt)
            plsc.store_scatter(cmp_r, [widx], win_r[pl.ds(j * 16, 16)])
            plsc.store_scatter(cmp_s, [widx], svec)
            return cur + plsc.all_reduce_population_count(m)[0]

        n_win = lax.fori_loop(0, SPW // 16, cmp_body, jnp.int32(0))
        nch = (n_win + C - 1) // C
        npad = nch * C - n_win

        last_off = jnp.maximum(n_win - 1, 0)
        last_t = jnp.full((16,), cmp_t[pl.ds(last_off, 16)][0], jnp.int32)
        last_r = jnp.full((16,), cmp_r[pl.ds(last_off, 16)][0], jnp.int32)
        last_s = jnp.full((16,), cmp_s[pl.ds(last_off, 16)][0], jnp.int32)
        for jk in range(C // 16):
            m = (iota + jk * 16) < npad
            pidx = jnp.where(m, n_win + jk * 16 + iota, DUMP)
            plsc.store_scatter(cmp_t, [pidx], last_t)
            plsc.store_scatter(cmp_r, [pidx], last_r)
            plsc.store_scatter(cmp_s, [pidx], last_s)

        for j in range(SPW // 16):
            svec = cmp_s[pl.ds(j * 16, 16)]
            row, col = j // (C // 16), (j % (C // 16)) * 16
            idx_s[row, pl.ds(col, 16)] = svec

        def data_body(c, _):
            co = c * C
            g1 = pltpu.async_copy(kv_hbm.at[cmp_t.at[pl.ds(co, C)]],
                                  row_buf.at[:, pl.ds(0, D)], gsem)
            g2 = pltpu.async_copy(score_hbm.at[cmp_t.at[pl.ds(co, C)]],
                                  row_buf.at[:, pl.ds(D, D)], gsem)
            g3 = pltpu.async_copy(ape_hbm.at[cmp_r.at[pl.ds(co, C)]],
                                  ape_buf, gsem)
            g1.wait()
            g2.wait()
            g3.wait()

            def add_body(i, _):
                for v in range(D // 16):
                    row_buf[i, pl.ds(D + v * 16, 16)] = (
                        row_buf[i, pl.ds(D + v * 16, 16)]
                        + ape_buf[i, pl.ds(v * 16, 16)])
                return 0

            lax.fori_loop(0, C, add_body, 0)

            s1 = pltpu.async_copy(row_buf, out_hbm.at[idx_s.at[c]], ssem)
            s1.wait()
            return 0

        lax.fori_loop(0, nch, data_body, 0)

    body(kv, score, ape, positions, slot_mapping, out_ref)


def kernel(kv, score, ape, state_cache, positions, slot_mapping):
    flat = state_cache.reshape(N_SLOTS, 2 * D)
    ref = jax.new_ref(flat)
    _sc_scatter(kv, score, ape, positions.astype(jnp.int32),
                slot_mapping.astype(jnp.int32), ref)
    return ref[...].reshape(state_cache.shape)
